# baseline (speedup 1.0000x reference)
import functools

import jax
import jax.numpy as jnp
from jax import lax
from jax.experimental import pallas as pl
from jax.experimental.pallas import tpu as pltpu
from jax.experimental.pallas import tpu_sc as plsc

_K = 16
_NW = 32


def _knn_kern(q_ref, bt_ref, o_ref, *, nb, bq, exclude_self, k):
    q = q_ref[...]
    bt = bt_ref[...]
    qn = (q[:, 0:1] * q[:, 0:1] + q[:, 1:2] * q[:, 1:2]) + q[:, 2:3] * q[:, 2:3]
    bn = (bt[0:1, :] * bt[0:1, :] + bt[1:2, :] * bt[1:2, :]) + bt[2:3, :] * bt[2:3, :]
    d = (qn + bn) - 2.0 * jnp.dot(q, bt, preferred_element_type=jnp.float32)
    col = lax.broadcasted_iota(jnp.int32, (bq, nb), 1)
    if exclude_self:
        row = lax.broadcasted_iota(jnp.int32, (bq, nb), 0) + pl.program_id(0) * bq
        d = jnp.where(col == row, jnp.float32(1e30), d)
    big = jnp.float32(1e30)
    m = jnp.min(d, axis=1, keepdims=True)
    idx = jnp.min(jnp.where(d == m, col, nb), axis=1, keepdims=True)
    outs = [idx]
    for _ in range(k - 1):
        valid = (d > m) | ((d == m) & (col > idx))
        m2 = jnp.min(jnp.where(valid, d, big), axis=1, keepdims=True)
        idx2 = jnp.min(
            jnp.where((d == m2) & ((m2 > m) | (col > idx)), col, nb),
            axis=1, keepdims=True)
        m, idx = m2, idx2
        outs.append(idx)
    o_ref[...] = jnp.concatenate(outs, axis=1)


def _knn(query, base, exclude_self):
    nq = query.shape[0]
    nb = base.shape[0]
    bq = min(nq, 256)
    bt = base.T
    return pl.pallas_call(
        functools.partial(_knn_kern, nb=nb, bq=bq, exclude_self=exclude_self, k=_K),
        grid=(nq // bq,),
        in_specs=[
            pl.BlockSpec((bq, 3), lambda i: (i, 0)),
            pl.BlockSpec((3, nb), lambda i: (0, 0)),
        ],
        out_specs=pl.BlockSpec((bq, _K), lambda i: (i, 0)),
        out_shape=jax.ShapeDtypeStruct((nq, _K), jnp.int32),
    )(query, bt)


def _fps_kern(pt_ref, pr_ref, o_ref, *, n, ns):
    n8 = n // 8
    px = pt_ref[0]
    py = pt_ref[1]
    pz = pt_ref[2]
    fid = (lax.broadcasted_iota(jnp.int32, (8, n8), 0) * n8
           + lax.broadcasted_iota(jnp.int32, (8, n8), 1))

    prow0 = pr_ref[0:1, :]
    o_ref[0:1, :] = prow0
    dx = px - prow0[:, 0:1]
    dy = py - prow0[:, 1:2]
    dz = pz - prow0[:, 2:3]
    mind0 = (dx * dx + dy * dy) + dz * dz

    def body(i, mind):
        m = jnp.max(mind)
        fsel = jnp.min(jnp.where(mind == m, fid, n))
        prow = pr_ref[pl.ds(fsel, 1), :]
        o_ref[pl.ds(i, 1), :] = prow
        ddx = px - prow[:, 0:1]
        ddy = py - prow[:, 1:2]
        ddz = pz - prow[:, 2:3]
        dd = (ddx * ddx + ddy * ddy) + ddz * ddz
        return jnp.minimum(mind, dd)

    lax.fori_loop(1, ns, body, mind0)


def _fps(pos, ns):
    n = pos.shape[0]
    pt = pos.T.reshape(3, 8, n // 8)
    return pl.pallas_call(
        functools.partial(_fps_kern, n=n, ns=ns),
        in_specs=[pl.BlockSpec((3, 8, n // 8), lambda: (0, 0, 0)),
                  pl.BlockSpec((n, 3), lambda: (0, 0))],
        out_specs=pl.BlockSpec((ns, 3), lambda: (0, 0)),
        out_shape=jax.ShapeDtypeStruct((ns, 3), jnp.float32),
    )(pt, pos)


def _mlpbn_kern(x_ref, w_ref, b_ref, g_ref, be_ref, o_ref, *, n, dout, dp):
    h = jnp.dot(x_ref[...], w_ref[...], preferred_element_type=jnp.float32) + b_ref[...]
    mu = jnp.mean(h, axis=0, keepdims=True)
    var = jnp.mean((h - mu) * (h - mu), axis=0, keepdims=True)
    hn = (h - mu) / jnp.sqrt(var + 1e-5) * g_ref[...] + be_ref[...]
    hn = jnp.maximum(hn, 0.0)
    if dp > dout:
        hn = jnp.concatenate([hn, jnp.zeros((n, dp - dout), jnp.float32)], axis=1)
    o_ref[...] = hn


def _mlp_bn(x, p, pad_to_128=False):
    n, din = x.shape
    dout = p['w'].shape[1]
    dp = max(dout, 128) if pad_to_128 else dout
    return pl.pallas_call(
        functools.partial(_mlpbn_kern, n=n, dout=dout, dp=dp),
        in_specs=[pl.BlockSpec((n, din), lambda: (0, 0)),
                  pl.BlockSpec((din, dout), lambda: (0, 0)),
                  pl.BlockSpec((1, dout), lambda: (0, 0)),
                  pl.BlockSpec((1, dout), lambda: (0, 0)),
                  pl.BlockSpec((1, dout), lambda: (0, 0))],
        out_specs=pl.BlockSpec((n, dp), lambda: (0, 0)),
        out_shape=jax.ShapeDtypeStruct((n, dp), jnp.float32),
    )(x, p['w'], p['b'].reshape(1, -1), p['gamma'].reshape(1, -1),
      p['beta'].reshape(1, -1))


def _sc_gather(table, idx):
    v, dd = table.shape
    b = idx.shape[0]
    bpw = b // _NW
    ch = 8
    while (ch * 2 <= min(bpw, 128)) and (ch * 2 * dd * 4 <= 393216):
        ch *= 2
    nch = bpw // ch
    mesh = plsc.VectorSubcoreMesh(core_axis_name="c", subcore_axis_name="s")

    @functools.partial(
        pl.kernel, mesh=mesh,
        out_type=jax.ShapeDtypeStruct((b, dd), jnp.float32),
        scratch_types=[
            pltpu.VMEM((ch,), jnp.int32),
            pltpu.VMEM((ch, dd), jnp.float32),
            pltpu.SemaphoreType.DMA,
        ],
    )
    def k(table_hbm, idx_hbm, out_hbm, idx_v, rows_v, sem):
        wid = lax.axis_index("s") * 2 + lax.axis_index("c")
        base = wid * bpw

        def body(ci, carry):
            off = base + ci * ch
            pltpu.sync_copy(idx_hbm.at[pl.ds(off, ch)], idx_v)
            pltpu.async_copy(table_hbm.at[idx_v], rows_v, sem).wait()
            pltpu.sync_copy(rows_v, out_hbm.at[pl.ds(off, ch)])
            return carry

        lax.fori_loop(0, nch, body, 0)

    return k(table, idx)


def _rowmax_kern(g_ref, o_ref, *, ns, dp, d):
    o_ref[...] = jnp.max(g_ref[...].reshape(ns, _K, dp), axis=1)[:, :d]


def _rowmax16(g, ns, d):
    dp = g.shape[1]
    return pl.pallas_call(
        functools.partial(_rowmax_kern, ns=ns, dp=dp, d=d),
        in_specs=[pl.BlockSpec((ns * _K, dp), lambda: (0, 0))],
        out_specs=pl.BlockSpec((ns, d), lambda: (0, 0)),
        out_shape=jax.ShapeDtypeStruct((ns, d), jnp.float32),
    )(g)


def _tw(d):
    return ((2 * d + 16) + 127) // 128 * 128


def _blockA_kern(x_ref, pos_ref, win_ref, bin_ref, wsrc_ref, wlin_ref,
                 wdst_ref, t_ref, adst_ref, *, n, d):
    h = jnp.maximum(
        jnp.dot(x_ref[...], win_ref[...], preferred_element_type=jnp.float32)
        + bin_ref[...], 0.0)
    asrc = jnp.dot(h, wsrc_ref[...], preferred_element_type=jnp.float32)
    vv = jnp.dot(h, wlin_ref[...], preferred_element_type=jnp.float32)
    pos_pad = jnp.concatenate(
        [pos_ref[...], jnp.zeros((n, _tw(d) - 2 * d - 3), jnp.float32)], axis=1)
    t_ref[...] = jnp.concatenate([asrc, vv, pos_pad], axis=1)
    adst_ref[...] = jnp.dot(h, wdst_ref[...], preferred_element_type=jnp.float32)


def _blockA(x, pos, prm):
    n, d = x.shape
    wp = _tw(d)
    return pl.pallas_call(
        functools.partial(_blockA_kern, n=n, d=d),
        in_specs=[pl.BlockSpec((n, d), lambda: (0, 0)),
                  pl.BlockSpec((n, 3), lambda: (0, 0)),
                  pl.BlockSpec((d, d), lambda: (0, 0)),
                  pl.BlockSpec((1, d), lambda: (0, 0)),
                  pl.BlockSpec((d, d), lambda: (0, 0)),
                  pl.BlockSpec((d, d), lambda: (0, 0)),
                  pl.BlockSpec((d, d), lambda: (0, 0))],
        out_specs=[pl.BlockSpec((n, wp), lambda: (0, 0)),
                   pl.BlockSpec((n, d), lambda: (0, 0))],
        out_shape=[jax.ShapeDtypeStruct((n, wp), jnp.float32),
                   jax.ShapeDtypeStruct((n, d), jnp.float32)],
    )(x, pos, prm['w_in'], prm['b_in'].reshape(1, -1), prm['w_src'],
      prm['w_lin'], prm['w_dst'])


def _blockB_kern(g_ref, t_ref, adst_ref, pos_ref, x_ref,
                 w1p_ref, b1p_ref, w2p_ref, b2p_ref,
                 w1a_ref, b1a_ref, w2a_ref, b2a_ref,
                 wout_ref, bout_ref, o_ref, *, bn, d):
    g = g_ref[...]
    asrc_g = g[:, :d]
    v_g = g[:, d:2 * d]
    pos_src = g[:, 2 * d:2 * d + 3]
    pos_dst = jnp.broadcast_to(
        pos_ref[...][:, None, :], (bn, _K, 3)).reshape(bn * _K, 3)
    pd = pos_dst - pos_src
    h1 = jnp.maximum(
        jnp.dot(pd, w1p_ref[...], preferred_element_type=jnp.float32)
        + b1p_ref[...], 0.0)
    delta = jnp.dot(h1, w2p_ref[...], preferred_element_type=jnp.float32) + b2p_ref[...]
    adst = adst_ref[...]
    adst_rep = jnp.broadcast_to(
        adst[:, None, :], (bn, _K, d)).reshape(bn * _K, d)
    att_in = adst_rep - asrc_g + delta
    a1 = jnp.maximum(
        jnp.dot(att_in, w1a_ref[...], preferred_element_type=jnp.float32)
        + b1a_ref[...], 0.0)
    alpha = jnp.dot(a1, w2a_ref[...], preferred_element_type=jnp.float32) + b2a_ref[...]
    h1s = jnp.maximum(b1p_ref[...], 0.0)
    delta_s = jnp.dot(h1s, w2p_ref[...], preferred_element_type=jnp.float32) + b2p_ref[...]
    att_in_s = adst - t_ref[...][:, :d] + delta_s
    a1s = jnp.maximum(
        jnp.dot(att_in_s, w1a_ref[...], preferred_element_type=jnp.float32)
        + b1a_ref[...], 0.0)
    alpha_s = jnp.dot(a1s, w2a_ref[...], preferred_element_type=jnp.float32) + b2a_ref[...]
    al3 = alpha.reshape(bn, _K, d)
    amax = jnp.maximum(jnp.max(al3, axis=1), alpha_s)
    ex3 = jnp.exp(al3 - amax[:, None, :])
    exs = jnp.exp(alpha_s - amax)
    den = jnp.sum(ex3, axis=1) + exs + 1e-16
    w3 = ex3 / den[:, None, :]
    v3 = v_g.reshape(bn, _K, d)
    dl3 = delta.reshape(bn, _K, d)
    msg = (jnp.sum(w3 * (v3 + dl3), axis=1)
           + (exs / den) * (t_ref[...][:, d:2 * d] + delta_s))
    out = jnp.maximum(
        jnp.dot(msg, wout_ref[...], preferred_element_type=jnp.float32)
        + bout_ref[...], 0.0) + x_ref[...]
    o_ref[...] = out


def _blockB(g, t, adst, pos, x, prm):
    n, d = x.shape
    wp = _tw(d)
    bn = min(n, 512)
    pnn = prm['pos_nn']
    ann = prm['attn_nn']
    full = lambda a, b: pl.BlockSpec((a, b), lambda i: (0, 0))
    return pl.pallas_call(
        functools.partial(_blockB_kern, bn=bn, d=d),
        grid=(n // bn,),
        in_specs=[
            pl.BlockSpec((bn * _K, wp), lambda i: (i, 0)),
            pl.BlockSpec((bn, wp), lambda i: (i, 0)),
            pl.BlockSpec((bn, d), lambda i: (i, 0)),
            pl.BlockSpec((bn, 3), lambda i: (i, 0)),
            pl.BlockSpec((bn, d), lambda i: (i, 0)),
            full(3, 64), full(1, 64), full(64, d), full(1, d),
            full(d, 64), full(1, 64), full(64, d), full(1, d),
            full(d, d), full(1, d),
        ],
        out_specs=pl.BlockSpec((bn, d), lambda i: (i, 0)),
        out_shape=jax.ShapeDtypeStruct((n, d), jnp.float32),
    )(g, t, adst, pos, x,
      pnn['w1'], pnn['b1'].reshape(1, -1), pnn['w2'], pnn['b2'].reshape(1, -1),
      ann['w1'], ann['b1'].reshape(1, -1), ann['w2'], ann['b2'].reshape(1, -1),
      prm['w_out'], prm['b_out'].reshape(1, -1))


def _ptblock(x, pos, nbr, prm):
    t, adst = _blockA(x, pos, prm)
    g = _sc_gather(t, nbr.reshape(-1))
    return _blockB(g, t, adst, pos, x, prm)


def _head_kern(x_ref, w1_ref, b1_ref, w2_ref, b2_ref, w3_ref, b3_ref, o_ref):
    xm = jnp.mean(x_ref[...], axis=0, keepdims=True)
    h = jnp.maximum(
        jnp.dot(xm, w1_ref[...], preferred_element_type=jnp.float32) + b1_ref[...], 0.0)
    h = jnp.maximum(
        jnp.dot(h, w2_ref[...], preferred_element_type=jnp.float32) + b2_ref[...], 0.0)
    o = jnp.dot(h, w3_ref[...], preferred_element_type=jnp.float32) + b3_ref[...]
    shifted = o - jnp.max(o)
    o_ref[...] = shifted - jnp.log(jnp.sum(jnp.exp(shifted)))


def _head(x, h):
    n, d = x.shape
    nc = h['w3'].shape[1]
    return pl.pallas_call(
        _head_kern,
        in_specs=[pl.BlockSpec((n, d), lambda: (0, 0)),
                  pl.BlockSpec((d, 64), lambda: (0, 0)),
                  pl.BlockSpec((1, 64), lambda: (0, 0)),
                  pl.BlockSpec((64, 64), lambda: (0, 0)),
                  pl.BlockSpec((1, 64), lambda: (0, 0)),
                  pl.BlockSpec((64, nc), lambda: (0, 0)),
                  pl.BlockSpec((1, nc), lambda: (0, 0))],
        out_specs=pl.BlockSpec((1, nc), lambda: (0, 0)),
        out_shape=jax.ShapeDtypeStruct((1, nc), jnp.float32),
    )(x, h['w1'], h['b1'].reshape(1, -1), h['w2'], h['b2'].reshape(1, -1),
      h['w3'], h['b3'].reshape(1, -1))


def kernel(x, pos, batch, params):
    del batch
    xx = _mlp_bn(x, params['mlp_input'])
    nbr0 = _knn(pos, pos, True)
    xx = _ptblock(xx, pos, nbr0, params['t_in'])
    p = pos
    n = pos.shape[0]
    for i in range(4):
        ns = n // 4
        selpos = _fps(p, ns)
        nbrd = _knn(selpos, p, False)
        ei = _knn(selpos, selpos, True)
        xm = _mlp_bn(xx, params['mlp_down'][i], pad_to_128=True)
        g = _sc_gather(xm, nbrd.reshape(-1))
        xs = _rowmax16(g, ns, params['mlp_down'][i]['w'].shape[1])
        xx = _ptblock(xs, selpos, ei, params['t_down'][i])
        p = selpos
        n = ns
    return _head(xx, params['head'])


# speedup vs baseline: 1.7162x; 1.7162x over previous
import functools

import jax
import jax.numpy as jnp
from jax import lax
from jax.experimental import pallas as pl
from jax.experimental.pallas import tpu as pltpu
from jax.experimental.pallas import tpu_sc as plsc

_K = 16
_NW = 32


def _knn_kern(q_ref, bt_ref, o_ref, *, nb, bq, exclude_self, k):
    q = q_ref[...]
    bt = bt_ref[...]
    qn = (q[:, 0:1] * q[:, 0:1] + q[:, 1:2] * q[:, 1:2]) + q[:, 2:3] * q[:, 2:3]
    bn = (bt[0:1, :] * bt[0:1, :] + bt[1:2, :] * bt[1:2, :]) + bt[2:3, :] * bt[2:3, :]
    d = (qn + bn) - 2.0 * jnp.dot(q, bt, preferred_element_type=jnp.float32)
    col = lax.broadcasted_iota(jnp.int32, (bq, nb), 1)
    if exclude_self:
        row = lax.broadcasted_iota(jnp.int32, (bq, nb), 0) + pl.program_id(0) * bq
        d = jnp.where(col == row, jnp.float32(1e30), d)
    big = jnp.float32(1e30)
    outs = []
    for _ in range(k):
        idx = jnp.argmin(d, axis=1).astype(jnp.int32)[:, None]
        outs.append(idx)
        d = jnp.where(col == idx, big, d)
    o_ref[...] = jnp.concatenate(outs, axis=1)


def _knn(query, base, exclude_self):
    nq = query.shape[0]
    nb = base.shape[0]
    bq = min(nq, 256)
    bt = base.T
    return pl.pallas_call(
        functools.partial(_knn_kern, nb=nb, bq=bq, exclude_self=exclude_self, k=_K),
        grid=(nq // bq,),
        in_specs=[
            pl.BlockSpec((bq, 3), lambda i: (i, 0)),
            pl.BlockSpec((3, nb), lambda i: (0, 0)),
        ],
        out_specs=pl.BlockSpec((bq, _K), lambda i: (i, 0)),
        out_shape=jax.ShapeDtypeStruct((nq, _K), jnp.int32),
    )(query, bt)


def _fps_kern(pt_ref, pr_ref, o_ref, *, n, ns):
    n8 = n // 8
    px = pt_ref[0]
    py = pt_ref[1]
    pz = pt_ref[2]
    fid = (lax.broadcasted_iota(jnp.int32, (8, n8), 0) * n8
           + lax.broadcasted_iota(jnp.int32, (8, n8), 1))

    prow0 = pr_ref[0:1, :]
    o_ref[0:1, :] = prow0
    dx = px - prow0[:, 0:1]
    dy = py - prow0[:, 1:2]
    dz = pz - prow0[:, 2:3]
    mind0 = (dx * dx + dy * dy) + dz * dz

    def body(i, mind):
        m = jnp.max(mind)
        fsel = jnp.min(jnp.where(mind == m, fid, n))
        prow = pr_ref[pl.ds(fsel, 1), :]
        o_ref[pl.ds(i, 1), :] = prow
        ddx = px - prow[:, 0:1]
        ddy = py - prow[:, 1:2]
        ddz = pz - prow[:, 2:3]
        dd = (ddx * ddx + ddy * ddy) + ddz * ddz
        return jnp.minimum(mind, dd)

    lax.fori_loop(1, ns, body, mind0)


def _fps(pos, ns):
    n = pos.shape[0]
    pt = pos.T.reshape(3, 8, n // 8)
    return pl.pallas_call(
        functools.partial(_fps_kern, n=n, ns=ns),
        in_specs=[pl.BlockSpec((3, 8, n // 8), lambda: (0, 0, 0)),
                  pl.BlockSpec((n, 3), lambda: (0, 0))],
        out_specs=pl.BlockSpec((ns, 3), lambda: (0, 0)),
        out_shape=jax.ShapeDtypeStruct((ns, 3), jnp.float32),
    )(pt, pos)


def _mlpbn_kern(x_ref, w_ref, b_ref, g_ref, be_ref, o_ref, *, n, dout, dp):
    h = jnp.dot(x_ref[...], w_ref[...], preferred_element_type=jnp.float32) + b_ref[...]
    mu = jnp.mean(h, axis=0, keepdims=True)
    var = jnp.mean((h - mu) * (h - mu), axis=0, keepdims=True)
    hn = (h - mu) / jnp.sqrt(var + 1e-5) * g_ref[...] + be_ref[...]
    hn = jnp.maximum(hn, 0.0)
    if dp > dout:
        hn = jnp.concatenate([hn, jnp.zeros((n, dp - dout), jnp.float32)], axis=1)
    o_ref[...] = hn


def _mlp_bn(x, p, pad_to_128=False):
    n, din = x.shape
    dout = p['w'].shape[1]
    dp = max(dout, 128) if pad_to_128 else dout
    return pl.pallas_call(
        functools.partial(_mlpbn_kern, n=n, dout=dout, dp=dp),
        in_specs=[pl.BlockSpec((n, din), lambda: (0, 0)),
                  pl.BlockSpec((din, dout), lambda: (0, 0)),
                  pl.BlockSpec((1, dout), lambda: (0, 0)),
                  pl.BlockSpec((1, dout), lambda: (0, 0)),
                  pl.BlockSpec((1, dout), lambda: (0, 0))],
        out_specs=pl.BlockSpec((n, dp), lambda: (0, 0)),
        out_shape=jax.ShapeDtypeStruct((n, dp), jnp.float32),
    )(x, p['w'], p['b'].reshape(1, -1), p['gamma'].reshape(1, -1),
      p['beta'].reshape(1, -1))


def _sc_gather(table, idx):
    v, dd = table.shape
    b = idx.shape[0]
    bpw = b // _NW
    ch = 8
    while (ch * 2 <= min(bpw, 128)) and (ch * 2 * dd * 4 <= 393216):
        ch *= 2
    nch = bpw // ch
    mesh = plsc.VectorSubcoreMesh(core_axis_name="c", subcore_axis_name="s")

    @functools.partial(
        pl.kernel, mesh=mesh,
        out_type=jax.ShapeDtypeStruct((b, dd), jnp.float32),
        scratch_types=[
            pltpu.VMEM((ch,), jnp.int32),
            pltpu.VMEM((ch, dd), jnp.float32),
            pltpu.SemaphoreType.DMA,
        ],
    )
    def k(table_hbm, idx_hbm, out_hbm, idx_v, rows_v, sem):
        wid = lax.axis_index("s") * 2 + lax.axis_index("c")
        base = wid * bpw

        def body(ci, carry):
            off = base + ci * ch
            pltpu.sync_copy(idx_hbm.at[pl.ds(off, ch)], idx_v)
            pltpu.async_copy(table_hbm.at[idx_v], rows_v, sem).wait()
            pltpu.sync_copy(rows_v, out_hbm.at[pl.ds(off, ch)])
            return carry

        lax.fori_loop(0, nch, body, 0)

    return k(table, idx)


def _rowmax_kern(g_ref, o_ref, *, ns, dp, d):
    o_ref[...] = jnp.max(g_ref[...].reshape(ns, _K, dp), axis=1)[:, :d]


def _rowmax16(g, ns, d):
    dp = g.shape[1]
    return pl.pallas_call(
        functools.partial(_rowmax_kern, ns=ns, dp=dp, d=d),
        in_specs=[pl.BlockSpec((ns * _K, dp), lambda: (0, 0))],
        out_specs=pl.BlockSpec((ns, d), lambda: (0, 0)),
        out_shape=jax.ShapeDtypeStruct((ns, d), jnp.float32),
    )(g)


def _tw(d):
    return ((2 * d + 16) + 127) // 128 * 128


def _blockA_kern(x_ref, pos_ref, win_ref, bin_ref, wsrc_ref, wlin_ref,
                 wdst_ref, t_ref, adst_ref, *, n, d):
    h = jnp.maximum(
        jnp.dot(x_ref[...], win_ref[...], preferred_element_type=jnp.float32)
        + bin_ref[...], 0.0)
    asrc = jnp.dot(h, wsrc_ref[...], preferred_element_type=jnp.float32)
    vv = jnp.dot(h, wlin_ref[...], preferred_element_type=jnp.float32)
    pos_pad = jnp.concatenate(
        [pos_ref[...], jnp.zeros((n, _tw(d) - 2 * d - 3), jnp.float32)], axis=1)
    t_ref[...] = jnp.concatenate([asrc, vv, pos_pad], axis=1)
    adst_ref[...] = jnp.dot(h, wdst_ref[...], preferred_element_type=jnp.float32)


def _blockA(x, pos, prm):
    n, d = x.shape
    wp = _tw(d)
    return pl.pallas_call(
        functools.partial(_blockA_kern, n=n, d=d),
        in_specs=[pl.BlockSpec((n, d), lambda: (0, 0)),
                  pl.BlockSpec((n, 3), lambda: (0, 0)),
                  pl.BlockSpec((d, d), lambda: (0, 0)),
                  pl.BlockSpec((1, d), lambda: (0, 0)),
                  pl.BlockSpec((d, d), lambda: (0, 0)),
                  pl.BlockSpec((d, d), lambda: (0, 0)),
                  pl.BlockSpec((d, d), lambda: (0, 0))],
        out_specs=[pl.BlockSpec((n, wp), lambda: (0, 0)),
                   pl.BlockSpec((n, d), lambda: (0, 0))],
        out_shape=[jax.ShapeDtypeStruct((n, wp), jnp.float32),
                   jax.ShapeDtypeStruct((n, d), jnp.float32)],
    )(x, pos, prm['w_in'], prm['b_in'].reshape(1, -1), prm['w_src'],
      prm['w_lin'], prm['w_dst'])


def _blockB_kern(g_ref, t_ref, adst_ref, pos_ref, x_ref,
                 w1p_ref, b1p_ref, w2p_ref, b2p_ref,
                 w1a_ref, b1a_ref, w2a_ref, b2a_ref,
                 wout_ref, bout_ref, o_ref, *, bn, d):
    g = g_ref[...]
    asrc_g = g[:, :d]
    v_g = g[:, d:2 * d]
    pos_src = g[:, 2 * d:2 * d + 3]
    pos_dst = jnp.broadcast_to(
        pos_ref[...][:, None, :], (bn, _K, 3)).reshape(bn * _K, 3)
    pd = pos_dst - pos_src
    h1 = jnp.maximum(
        jnp.dot(pd, w1p_ref[...], preferred_element_type=jnp.float32)
        + b1p_ref[...], 0.0)
    delta = jnp.dot(h1, w2p_ref[...], preferred_element_type=jnp.float32) + b2p_ref[...]
    adst = adst_ref[...]
    adst_rep = jnp.broadcast_to(
        adst[:, None, :], (bn, _K, d)).reshape(bn * _K, d)
    att_in = adst_rep - asrc_g + delta
    a1 = jnp.maximum(
        jnp.dot(att_in, w1a_ref[...], preferred_element_type=jnp.float32)
        + b1a_ref[...], 0.0)
    alpha = jnp.dot(a1, w2a_ref[...], preferred_element_type=jnp.float32) + b2a_ref[...]
    h1s = jnp.maximum(b1p_ref[...], 0.0)
    delta_s = jnp.dot(h1s, w2p_ref[...], preferred_element_type=jnp.float32) + b2p_ref[...]
    att_in_s = adst - t_ref[...][:, :d] + delta_s
    a1s = jnp.maximum(
        jnp.dot(att_in_s, w1a_ref[...], preferred_element_type=jnp.float32)
        + b1a_ref[...], 0.0)
    alpha_s = jnp.dot(a1s, w2a_ref[...], preferred_element_type=jnp.float32) + b2a_ref[...]
    al3 = alpha.reshape(bn, _K, d)
    amax = jnp.maximum(jnp.max(al3, axis=1), alpha_s)
    ex3 = jnp.exp(al3 - amax[:, None, :])
    exs = jnp.exp(alpha_s - amax)
    den = jnp.sum(ex3, axis=1) + exs + 1e-16
    w3 = ex3 / den[:, None, :]
    v3 = v_g.reshape(bn, _K, d)
    dl3 = delta.reshape(bn, _K, d)
    msg = (jnp.sum(w3 * (v3 + dl3), axis=1)
           + (exs / den) * (t_ref[...][:, d:2 * d] + delta_s))
    out = jnp.maximum(
        jnp.dot(msg, wout_ref[...], preferred_element_type=jnp.float32)
        + bout_ref[...], 0.0) + x_ref[...]
    o_ref[...] = out


def _blockB(g, t, adst, pos, x, prm):
    n, d = x.shape
    wp = _tw(d)
    bn = min(n, 512)
    pnn = prm['pos_nn']
    ann = prm['attn_nn']
    full = lambda a, b: pl.BlockSpec((a, b), lambda i: (0, 0))
    return pl.pallas_call(
        functools.partial(_blockB_kern, bn=bn, d=d),
        grid=(n // bn,),
        in_specs=[
            pl.BlockSpec((bn * _K, wp), lambda i: (i, 0)),
            pl.BlockSpec((bn, wp), lambda i: (i, 0)),
            pl.BlockSpec((bn, d), lambda i: (i, 0)),
            pl.BlockSpec((bn, 3), lambda i: (i, 0)),
            pl.BlockSpec((bn, d), lambda i: (i, 0)),
            full(3, 64), full(1, 64), full(64, d), full(1, d),
            full(d, 64), full(1, 64), full(64, d), full(1, d),
            full(d, d), full(1, d),
        ],
        out_specs=pl.BlockSpec((bn, d), lambda i: (i, 0)),
        out_shape=jax.ShapeDtypeStruct((n, d), jnp.float32),
    )(g, t, adst, pos, x,
      pnn['w1'], pnn['b1'].reshape(1, -1), pnn['w2'], pnn['b2'].reshape(1, -1),
      ann['w1'], ann['b1'].reshape(1, -1), ann['w2'], ann['b2'].reshape(1, -1),
      prm['w_out'], prm['b_out'].reshape(1, -1))


def _ptblock(x, pos, nbr, prm):
    t, adst = _blockA(x, pos, prm)
    g = _sc_gather(t, nbr.reshape(-1))
    return _blockB(g, t, adst, pos, x, prm)


def _head_kern(x_ref, w1_ref, b1_ref, w2_ref, b2_ref, w3_ref, b3_ref, o_ref):
    xm = jnp.mean(x_ref[...], axis=0, keepdims=True)
    h = jnp.maximum(
        jnp.dot(xm, w1_ref[...], preferred_element_type=jnp.float32) + b1_ref[...], 0.0)
    h = jnp.maximum(
        jnp.dot(h, w2_ref[...], preferred_element_type=jnp.float32) + b2_ref[...], 0.0)
    o = jnp.dot(h, w3_ref[...], preferred_element_type=jnp.float32) + b3_ref[...]
    shifted = o - jnp.max(o)
    o_ref[...] = shifted - jnp.log(jnp.sum(jnp.exp(shifted)))


def _head(x, h):
    n, d = x.shape
    nc = h['w3'].shape[1]
    return pl.pallas_call(
        _head_kern,
        in_specs=[pl.BlockSpec((n, d), lambda: (0, 0)),
                  pl.BlockSpec((d, 64), lambda: (0, 0)),
                  pl.BlockSpec((1, 64), lambda: (0, 0)),
                  pl.BlockSpec((64, 64), lambda: (0, 0)),
                  pl.BlockSpec((1, 64), lambda: (0, 0)),
                  pl.BlockSpec((64, nc), lambda: (0, 0)),
                  pl.BlockSpec((1, nc), lambda: (0, 0))],
        out_specs=pl.BlockSpec((1, nc), lambda: (0, 0)),
        out_shape=jax.ShapeDtypeStruct((1, nc), jnp.float32),
    )(x, h['w1'], h['b1'].reshape(1, -1), h['w2'], h['b2'].reshape(1, -1),
      h['w3'], h['b3'].reshape(1, -1))


def kernel(x, pos, batch, params):
    del batch
    xx = _mlp_bn(x, params['mlp_input'])
    nbr0 = _knn(pos, pos, True)
    xx = _ptblock(xx, pos, nbr0, params['t_in'])
    p = pos
    n = pos.shape[0]
    for i in range(4):
        ns = n // 4
        selpos = _fps(p, ns)
        nbrd = _knn(selpos, p, False)
        ei = _knn(selpos, selpos, True)
        xm = _mlp_bn(xx, params['mlp_down'][i], pad_to_128=True)
        g = _sc_gather(xm, nbrd.reshape(-1))
        xs = _rowmax16(g, ns, params['mlp_down'][i]['w'].shape[1])
        xx = _ptblock(xs, selpos, ei, params['t_down'][i])
        p = selpos
        n = ns
    return _head(xx, params['head'])
